# baseline (speedup 1.0000x reference)
import functools
import jax
import jax.numpy as jnp
from jax import lax
from jax.experimental import pallas as pl
from jax.experimental.pallas import tpu as pltpu
from jax.experimental.pallas import tpu_sc as plsc

NC = 2
NS = 16
NW = NC * NS
R = 16


def kernel(x, pos_table):
    batch, seq_len, embed = x.shape
    rows_per_w = seq_len // NW
    n_chunks = rows_per_w // R
    n_slices = embed // 16

    mesh = plsc.VectorSubcoreMesh(core_axis_name="c", subcore_axis_name="s")

    @functools.partial(
        pl.kernel,
        out_type=jax.ShapeDtypeStruct((batch, seq_len, embed), jnp.float32),
        mesh=mesh,
        scratch_types=[
            pltpu.VMEM((R, embed), jnp.float32),
            pltpu.VMEM((R, embed), jnp.float32),
        ],
    )
    def sc_add(x_hbm, pos_hbm, out_hbm, pos_v, x_v):
        wid = lax.axis_index("s") * NC + lax.axis_index("c")
        base = wid * rows_per_w

        def chunk_body(k, _):
            row0 = base + k * R
            pltpu.sync_copy(pos_hbm.at[pl.ds(row0, R)], pos_v)

            def batch_body(b, _):
                pltpu.sync_copy(x_hbm.at[b, pl.ds(row0, R)], x_v)

                def row_body(r, _):
                    for c in range(n_slices):
                        sl = pl.ds(c * 16, 16)
                        x_v[r, sl] = x_v[r, sl] + pos_v[r, sl]
                    return 0

                lax.fori_loop(0, R, row_body, 0)
                pltpu.sync_copy(x_v, out_hbm.at[b, pl.ds(row0, R)])
                return 0

            lax.fori_loop(0, batch, batch_body, 0)
            return 0

        lax.fori_loop(0, n_chunks, chunk_body, 0)

    return sc_add(x, pos_table)


# speedup vs baseline: 1.2668x; 1.2668x over previous
import functools
import jax
import jax.numpy as jnp
from jax import lax
from jax.experimental import pallas as pl
from jax.experimental.pallas import tpu as pltpu
from jax.experimental.pallas import tpu_sc as plsc

NC = 2
NS = 16
NW = NC * NS
R = 16


def kernel(x, pos_table):
    batch, seq_len, embed = x.shape
    rows_per_w = seq_len // NW
    n_chunks = rows_per_w // R
    n_slices = embed // 16

    mesh = plsc.VectorSubcoreMesh(core_axis_name="c", subcore_axis_name="s")

    @functools.partial(
        pl.kernel,
        out_type=jax.ShapeDtypeStruct((batch, seq_len, embed), jnp.float32),
        mesh=mesh,
        scratch_types=[
            pltpu.VMEM((2, R, embed), jnp.float32),
            pltpu.VMEM((batch, R, embed), jnp.float32),
            pltpu.SemaphoreType.DMA((2,)),
            pltpu.SemaphoreType.DMA((batch,)),
            pltpu.SemaphoreType.DMA((batch,)),
        ],
    )
    def sc_add(x_hbm, pos_hbm, out_hbm, pos_v, x_v, psem, xsem, ssem):
        wid = lax.axis_index("s") * NC + lax.axis_index("c")
        base = wid * rows_per_w

        def do_task(k, row0, j, b, first_chunk, last_chunk):
            pltpu.make_async_copy(
                x_hbm.at[b, pl.ds(row0, R)], x_v.at[b], xsem.at[b]
            ).wait()

            bn = (b + 1) % batch
            if b < batch - 1:
                kn_row = row0
            elif last_chunk:
                kn_row = row0
            else:
                kn_row = row0 + R
            if not (first_chunk and b < batch - 1):
                pltpu.make_async_copy(
                    x_v.at[bn], out_hbm.at[bn, pl.ds(kn_row, R)], ssem.at[bn]
                ).wait()
            pltpu.async_copy(
                x_hbm.at[bn, pl.ds(kn_row, R)], x_v.at[bn], xsem.at[bn]
            )

            def row_body(r, _):
                for c in range(n_slices):
                    sl = pl.ds(c * 16, 16)
                    x_v[b, r, sl] = x_v[b, r, sl] + pos_v[j, r, sl]
                return 0

            lax.fori_loop(0, R, row_body, 0)

            pltpu.async_copy(
                x_v.at[b], out_hbm.at[b, pl.ds(row0, R)], ssem.at[b]
            )

        def do_chunk(k, j, first_chunk=False, last_chunk=False):
            row0 = base + k * R
            pltpu.make_async_copy(
                pos_hbm.at[pl.ds(row0, R)], pos_v.at[j], psem.at[j]
            ).wait()
            rowp = row0 if last_chunk else row0 + R
            pltpu.async_copy(
                pos_hbm.at[pl.ds(rowp, R)], pos_v.at[1 - j], psem.at[1 - j]
            )
            for b in range(batch):
                do_task(k, row0, j, b, first_chunk, last_chunk)

        pltpu.async_copy(pos_hbm.at[pl.ds(base, R)], pos_v.at[0], psem.at[0])
        pltpu.async_copy(x_hbm.at[0, pl.ds(base, R)], x_v.at[0], xsem.at[0])

        do_chunk(0, 0, first_chunk=True)

        def kk_body(kk, _):
            k = 1 + 2 * kk
            do_chunk(k, 1)
            do_chunk(k + 1, 0)
            return 0

        lax.fori_loop(0, (n_chunks - 2) // 2, kk_body, 0)

        do_chunk(n_chunks - 1, 1, last_chunk=True)

        for b in range(1, batch):
            pltpu.make_async_copy(
                x_v.at[b], out_hbm.at[b, pl.ds(base, R)], ssem.at[b]
            ).wait()
        pltpu.make_async_copy(
            x_hbm.at[0, pl.ds(base, R)], x_v.at[0], xsem.at[0]
        ).wait()
        pltpu.make_async_copy(
            pos_hbm.at[pl.ds(base, R)], pos_v.at[0], psem.at[0]
        ).wait()

    return sc_add(x, pos_table)


# speedup vs baseline: 1.5397x; 1.2154x over previous
import functools
import jax
import jax.numpy as jnp
from jax import lax
from jax.experimental import pallas as pl
from jax.experimental.pallas import tpu as pltpu
from jax.experimental.pallas import tpu_sc as plsc

NC = 2
NS = 16
NW = NC * NS
R = 8


def kernel(x, pos_table):
    batch, seq_len, embed = x.shape
    rows_per_w = seq_len // NW
    n_chunks = rows_per_w // R
    n_slices = embed // 16

    mesh = plsc.VectorSubcoreMesh(core_axis_name="c", subcore_axis_name="s")

    @functools.partial(
        pl.kernel,
        out_type=jax.ShapeDtypeStruct((batch, seq_len, embed), jnp.float32),
        mesh=mesh,
        scratch_types=[
            pltpu.VMEM((2, R, embed), jnp.float32),
            pltpu.VMEM((batch, R, embed), jnp.float32),
            pltpu.VMEM((2, R, embed), jnp.float32),
            pltpu.SemaphoreType.DMA((2,)),
            pltpu.SemaphoreType.DMA((batch,)),
            pltpu.SemaphoreType.DMA((2,)),
        ],
    )
    def sc_add(x_hbm, pos_hbm, out_hbm, pos_v, x_v, o_v, psem, xsem, ssem):
        wid = lax.axis_index("s") * NC + lax.axis_index("c")
        base = wid * rows_per_w

        def do_task(row0, j, b, first_chunk, last_chunk):
            m = b % 2
            pltpu.make_async_copy(
                x_hbm.at[b, pl.ds(row0, R)], x_v.at[b], xsem.at[b]
            ).wait()

            bn = (b + 1) % batch
            if b < batch - 1 or last_chunk:
                rown = row0
            else:
                rown = row0 + R
            pltpu.async_copy(
                x_hbm.at[bn, pl.ds(rown, R)], x_v.at[bn], xsem.at[bn]
            )

            if not (first_chunk and b < 2):
                pltpu.make_async_copy(
                    o_v.at[m], out_hbm.at[b, pl.ds(row0, R)], ssem.at[m]
                ).wait()

            def row_body(r, _):
                for c in range(n_slices):
                    sl = pl.ds(c * 16, 16)
                    o_v[m, r, sl] = x_v[b, r, sl] + pos_v[j, r, sl]
                return 0

            lax.fori_loop(0, R, row_body, 0)

            pltpu.async_copy(
                o_v.at[m], out_hbm.at[b, pl.ds(row0, R)], ssem.at[m]
            )

        def do_chunk(k, j, first_chunk=False, last_chunk=False):
            row0 = base + k * R
            pltpu.make_async_copy(
                pos_hbm.at[pl.ds(row0, R)], pos_v.at[j], psem.at[j]
            ).wait()
            rowp = row0 if last_chunk else row0 + R
            pltpu.async_copy(
                pos_hbm.at[pl.ds(rowp, R)], pos_v.at[1 - j], psem.at[1 - j]
            )
            for b in range(batch):
                do_task(row0, j, b, first_chunk, last_chunk)

        pltpu.async_copy(pos_hbm.at[pl.ds(base, R)], pos_v.at[0], psem.at[0])
        pltpu.async_copy(x_hbm.at[0, pl.ds(base, R)], x_v.at[0], xsem.at[0])

        do_chunk(0, 0, first_chunk=True)

        def kk_body(kk, _):
            k = 1 + 2 * kk
            do_chunk(k, 1)
            do_chunk(k + 1, 0)
            return 0

        lax.fori_loop(0, (n_chunks - 2) // 2, kk_body, 0)

        do_chunk(n_chunks - 1, 1, last_chunk=True)

        for m in range(2):
            pltpu.make_async_copy(
                o_v.at[m], out_hbm.at[0, pl.ds(base, R)], ssem.at[m]
            ).wait()
        pltpu.make_async_copy(
            x_hbm.at[0, pl.ds(base, R)], x_v.at[0], xsem.at[0]
        ).wait()
        pltpu.make_async_copy(
            pos_hbm.at[pl.ds(base, R)], pos_v.at[0], psem.at[0]
        ).wait()

    return sc_add(x, pos_table)


# speedup vs baseline: 3.1001x; 2.0135x over previous
import jax
import jax.numpy as jnp
from jax import lax
from jax.experimental import pallas as pl
from jax.experimental.pallas import tpu as pltpu

S = 1024


def kernel(x, pos_table):
    batch, seq_len, embed = x.shape
    n_chunks = seq_len // S
    n_tasks = n_chunks * batch

    def body(x_hbm, pos_hbm, out_hbm, x_v, pos_v, o_v, xsem, psem, ssem):
        def issue_xload(t):
            k, b = t // batch, t % batch
            pltpu.async_copy(
                x_hbm.at[b, pl.ds(k * S, S)], x_v.at[t % 3], xsem.at[t % 3]
            )

        pltpu.async_copy(pos_hbm.at[pl.ds(0, S)], pos_v.at[0], psem.at[0])
        issue_xload(0)
        issue_xload(1)

        def task(t, _):
            k, b = t // batch, t % batch

            @pl.when(b == 0)
            def _():
                pltpu.make_async_copy(
                    pos_hbm.at[pl.ds(k * S, S)], pos_v.at[k % 2], psem.at[k % 2]
                ).wait()

                @pl.when(k < n_chunks - 1)
                def _():
                    pltpu.async_copy(
                        pos_hbm.at[pl.ds((k + 1) * S, S)],
                        pos_v.at[(k + 1) % 2],
                        psem.at[(k + 1) % 2],
                    )

            @pl.when(t + 2 < n_tasks)
            def _():
                tn = t + 2
                kn, bn = tn // batch, tn % batch
                pltpu.async_copy(
                    x_hbm.at[bn, pl.ds(kn * S, S)], x_v.at[tn % 3], xsem.at[tn % 3]
                )

            pltpu.make_async_copy(
                x_hbm.at[b, pl.ds(k * S, S)], x_v.at[t % 3], xsem.at[t % 3]
            ).wait()

            @pl.when(t >= 3)
            def _():
                pltpu.make_async_copy(
                    o_v.at[t % 3], out_hbm.at[b, pl.ds(k * S, S)], ssem.at[t % 3]
                ).wait()

            o_v[t % 3] = x_v[t % 3] + pos_v[k % 2]

            pltpu.async_copy(
                o_v.at[t % 3], out_hbm.at[b, pl.ds(k * S, S)], ssem.at[t % 3]
            )
            return 0

        lax.fori_loop(0, n_tasks, task, 0)

        for sl in range(3):
            pltpu.make_async_copy(
                o_v.at[sl], out_hbm.at[0, pl.ds(0, S)], ssem.at[sl]
            ).wait()

    return pl.pallas_call(
        body,
        in_specs=[
            pl.BlockSpec(memory_space=pl.ANY),
            pl.BlockSpec(memory_space=pl.ANY),
        ],
        out_specs=pl.BlockSpec(memory_space=pl.ANY),
        out_shape=jax.ShapeDtypeStruct((batch, seq_len, embed), x.dtype),
        scratch_shapes=[
            pltpu.VMEM((3, S, embed), jnp.float32),
            pltpu.VMEM((2, S, embed), jnp.float32),
            pltpu.VMEM((3, S, embed), jnp.float32),
            pltpu.SemaphoreType.DMA((3,)),
            pltpu.SemaphoreType.DMA((2,)),
            pltpu.SemaphoreType.DMA((3,)),
        ],
    )(x, pos_table[:seq_len])


# speedup vs baseline: 3.1036x; 1.0011x over previous
import jax
import jax.numpy as jnp
from jax.experimental import pallas as pl

BLK_S = 2048


def _add_kernel(x_ref, pos_ref, o_ref):
    o_ref[...] = x_ref[...] + pos_ref[...]


def kernel(x, pos_table):
    batch, seq_len, embed = x.shape
    grid = (seq_len // BLK_S, batch)
    return pl.pallas_call(
        _add_kernel,
        grid=grid,
        in_specs=[
            pl.BlockSpec((1, BLK_S, embed), lambda s, b: (b, s, 0)),
            pl.BlockSpec((BLK_S, embed), lambda s, b: (s, 0)),
        ],
        out_specs=pl.BlockSpec((1, BLK_S, embed), lambda s, b: (b, s, 0)),
        out_shape=jax.ShapeDtypeStruct((batch, seq_len, embed), x.dtype),
    )(x, pos_table[:seq_len])
